# Initial kernel scaffold; baseline (speedup 1.0000x reference)
import jax
import jax.numpy as jnp
from jax.experimental import pallas as pl


def kernel(x, w1q, b1s, wc2, b2s, wf0, wf1, bf, s0, s1, wcon, bcon, wfl, bfl, wout, bout):
    raise NotImplementedError("write your pallas kernel here")


# speedup vs baseline: 1.1233x; 1.1233x over previous
import jax
import jax.numpy as jnp
from jax.experimental import pallas as pl
from jax.experimental.pallas import tpu as pltpu

L_IN = 10178
PH = 120
M_IN = 87
U1 = 24
MC1 = 85
MP1 = 82
W4 = 81
NEG = -1e30


def _ae_kernel(x_ref, w1_ref, b1_ref, wc2_ref, b2_ref,
               wf0_ref, wf1_ref, bf_ref, s0_ref, s1_ref,
               z_ref, x44_ref):
    x = x_ref[0]
    xs = jnp.concatenate([x[:, 0:MC1], x[:, 1:1 + MC1], x[:, 2:2 + MC1]], axis=0)
    c1 = jnp.dot(w1_ref[...], xs, preferred_element_type=jnp.float32) + b1_ref[...]

    pad = jnp.full((U1 * 16, 3), NEG, jnp.float32)
    m = jnp.concatenate([c1, pad], axis=1)
    for s in (1, 2, 4, 8, 16, 32):
        a0, r = s // U1, s % U1
        sh = jnp.concatenate([m[16 * r:U1 * 16, a0:a0 + MC1],
                              m[0:16 * r, a0 + 1:a0 + 1 + MC1]], axis=0)
        m = jnp.concatenate([jnp.maximum(m[:, 0:MC1], sh), pad], axis=1)
    p1 = jnp.concatenate([m[0:16, 0:MP1], m[48:64, 0:MP1],
                          m[192:208, 0:MP1], m[240:256, 0:MP1]], axis=0)
    p1 = jnp.maximum(p1, 0.0).astype(jnp.bfloat16)

    c2 = jnp.dot(wc2_ref[...], p1, preferred_element_type=jnp.float32) + b2_ref[...]
    z = jnp.maximum(jnp.maximum(c2[0:32, 0:W4], c2[32:64, 0:W4]), c2[0:32, 1:W4 + 1])
    z = jnp.maximum(z, 0.0)
    zb = z.astype(jnp.bfloat16)
    z_ref[0] = zb

    t0 = jnp.dot(zb, s0_ref[...], preferred_element_type=jnp.float32)
    t1 = jnp.dot(zb, s1_ref[...], preferred_element_type=jnp.float32)
    slab = (jnp.dot(wf0_ref[...], t0.astype(jnp.bfloat16),
                    preferred_element_type=jnp.float32)
            + jnp.dot(wf1_ref[...], t1.astype(jnp.bfloat16),
                      preferred_element_type=jnp.float32)
            + bf_ref[...])
    x44_ref[0] = jnp.maximum(slab, 0.0).astype(jnp.bfloat16)


def _head_kernel(zf_ref, x44f_ref, wcon_ref, bcon_ref, wfl_ref, bfl_ref,
                 wout_ref, bout_ref, out_ref, x6_ref):
    B = x44f_ref.shape[0]
    x5 = jnp.dot(zf_ref[...], wcon_ref[...],
                 preferred_element_type=jnp.float32) + bcon_ref[...]
    x5 = jnp.maximum(x5, 0.0)
    x54 = jnp.dot(x44f_ref[...], wfl_ref[...],
                  preferred_element_type=jnp.float32) + bfl_ref[...]
    x54 = jnp.maximum(x54, 0.0)
    x6 = jnp.concatenate([x5[0:B], x5[B:2 * B], x5[2 * B:3 * B], x54], axis=1)
    x6_ref[...] = x6
    out_ref[...] = jnp.dot(x6.astype(jnp.bfloat16), wout_ref[...],
                           preferred_element_type=jnp.float32) + bout_ref[...]


def kernel(x, w1q, b1s, wc2, b2s, wf0, wf1, bf, s0, s1,
           wcon, bcon, wfl, bfl, wout, bout):
    B, C, S, L = x.shape
    NS = 3 * B
    bf16 = jnp.bfloat16
    xb = x[:, 0].astype(bf16)
    pairs = jnp.stack([xb[:, 0:2], xb[:, 2:4], xb[:, 4:6]], axis=0).reshape(NS, 2, L)
    xpad = jnp.pad(pairs, ((0, 0), (0, 0), (0, PH * M_IN - L)))
    x120 = xpad.reshape(NS, 2, M_IN, PH).transpose(0, 1, 3, 2).reshape(NS, 2 * PH, M_IN)

    w1 = jnp.concatenate([w1q[0], w1q[1], w1q[2]], axis=1).astype(bf16)
    wc2b, wf0b, wf1b = wc2.astype(bf16), wf0.astype(bf16), wf1.astype(bf16)
    s0b, s1b = s0.astype(bf16), s1.astype(bf16)
    nc = wout.shape[1]

    z, x44s = pl.pallas_call(
        _ae_kernel,
        out_shape=(jax.ShapeDtypeStruct((NS, 32, W4), bf16),
                   jax.ShapeDtypeStruct((NS, 32, 27), bf16)),
        grid=(NS,),
        in_specs=[
            pl.BlockSpec((1, 2 * PH, M_IN), lambda g: (g, 0, 0)),
            pl.BlockSpec((U1 * 16, 3 * 2 * PH), lambda g: (0, 0)),
            pl.BlockSpec((U1 * 16, 1), lambda g: (0, 0)),
            pl.BlockSpec((64, 64), lambda g: (0, 0)),
            pl.BlockSpec((64, 1), lambda g: (0, 0)),
            pl.BlockSpec((32, 32), lambda g: (0, 0)),
            pl.BlockSpec((32, 32), lambda g: (0, 0)),
            pl.BlockSpec((32, 1), lambda g: (0, 0)),
            pl.BlockSpec((W4, 27), lambda g: (0, 0)),
            pl.BlockSpec((W4, 27), lambda g: (0, 0)),
        ],
        out_specs=(pl.BlockSpec((1, 32, W4), lambda g: (g, 0, 0)),
                   pl.BlockSpec((1, 32, 27), lambda g: (g, 0, 0))),
        compiler_params=pltpu.CompilerParams(dimension_semantics=("parallel",)),
    )(x120, w1, b1s, wc2b, b2s, wf0b, wf1b, bf, s0b, s1b)

    z_flat = z.reshape(NS, 32 * W4)
    x44_flat = x44s.reshape(3, B, 32, 27).transpose(1, 2, 0, 3).reshape(B, 32 * W4)

    out, x6 = pl.pallas_call(
        _head_kernel,
        out_shape=(jax.ShapeDtypeStruct((B, nc), jnp.float32),
                   jax.ShapeDtypeStruct((B, 128), jnp.float32)),
        grid=(1,),
        in_specs=[
            pl.BlockSpec((NS, 32 * W4), lambda i: (0, 0)),
            pl.BlockSpec((B, 32 * W4), lambda i: (0, 0)),
            pl.BlockSpec((32 * W4, 32), lambda i: (0, 0)),
            pl.BlockSpec((1, 32), lambda i: (0, 0)),
            pl.BlockSpec((32 * W4, 32), lambda i: (0, 0)),
            pl.BlockSpec((1, 32), lambda i: (0, 0)),
            pl.BlockSpec((128, nc), lambda i: (0, 0)),
            pl.BlockSpec((1, nc), lambda i: (0, 0)),
        ],
        out_specs=(pl.BlockSpec((B, nc), lambda i: (0, 0)),
                   pl.BlockSpec((B, 128), lambda i: (0, 0))),
    )(z_flat, x44_flat, wcon.astype(bf16), bcon, wfl.astype(bf16), bfl,
      wout.astype(bf16), bout)
    return out, x6


# speedup vs baseline: 1.2770x; 1.1368x over previous
import jax
import jax.numpy as jnp
from jax.experimental import pallas as pl
from jax.experimental.pallas import tpu as pltpu

L_IN = 10178
PH = 120
M_IN = 87
U1 = 24
MC1 = 85
MP1 = 82
W4 = 81
NEG = -1e30


G = 4
_POOL_W = ((1, 88), (2, 87), (4, 86), (8, 85), (16, 84), (32, 82))


def _ae_kernel(x_ref, w1_ref, b1_ref, wc2_ref, b2_ref,
               wf0_ref, wf1_ref, bf_ref, s0_ref, s1_ref,
               z_ref, x44_ref):
    for i in range(G):
        x = x_ref[i]
        xw = jnp.concatenate([x, jnp.zeros((2 * PH, 4), x.dtype)], axis=1)
        xs = jnp.concatenate([xw[:, 0:89], xw[:, 1:90], xw[:, 2:91]], axis=0)
        m = jnp.dot(w1_ref[...], xs, preferred_element_type=jnp.float32) + b1_ref[...]

        for s, w in _POOL_W:
            a0, r = s // U1, s % U1
            sh = jnp.concatenate([m[16 * r:U1 * 16, a0:a0 + w],
                                  m[0:16 * r, a0 + 1:a0 + 1 + w]], axis=0)
            m = jnp.maximum(m[:, 0:w], sh)
        p1 = jnp.concatenate([m[0:16], m[48:64], m[192:208], m[240:256]], axis=0)
        p1 = jnp.maximum(p1, 0.0).astype(jnp.bfloat16)

        c2 = jnp.dot(wc2_ref[...], p1, preferred_element_type=jnp.float32) + b2_ref[...]
        z = jnp.maximum(jnp.maximum(c2[0:32, 0:W4], c2[32:64, 0:W4]),
                        c2[0:32, 1:W4 + 1])
        z = jnp.maximum(z, 0.0)
        zb = z.astype(jnp.bfloat16)
        z_ref[i] = zb

        t0 = jnp.dot(zb, s0_ref[...], preferred_element_type=jnp.float32)
        t1 = jnp.dot(zb, s1_ref[...], preferred_element_type=jnp.float32)
        slab = (jnp.dot(wf0_ref[...], t0.astype(jnp.bfloat16),
                        preferred_element_type=jnp.float32)
                + jnp.dot(wf1_ref[...], t1.astype(jnp.bfloat16),
                          preferred_element_type=jnp.float32)
                + bf_ref[...])
        x44_ref[i] = jnp.maximum(slab, 0.0).astype(jnp.bfloat16)


def _head_kernel(zf_ref, x44f_ref, wcon_ref, bcon_ref, wfl_ref, bfl_ref,
                 wout_ref, bout_ref, out_ref, x6_ref):
    B = x44f_ref.shape[0]
    x5 = jnp.dot(zf_ref[...], wcon_ref[...],
                 preferred_element_type=jnp.float32) + bcon_ref[...]
    x5 = jnp.maximum(x5, 0.0)
    x54 = jnp.dot(x44f_ref[...], wfl_ref[...],
                  preferred_element_type=jnp.float32) + bfl_ref[...]
    x54 = jnp.maximum(x54, 0.0)
    x6 = jnp.concatenate([x5[0:B], x5[B:2 * B], x5[2 * B:3 * B], x54], axis=1)
    x6_ref[...] = x6
    out_ref[...] = jnp.dot(x6.astype(jnp.bfloat16), wout_ref[...],
                           preferred_element_type=jnp.float32) + bout_ref[...]


def kernel(x, w1q, b1s, wc2, b2s, wf0, wf1, bf, s0, s1,
           wcon, bcon, wfl, bfl, wout, bout):
    B, C, S, L = x.shape
    NS = 3 * B
    bf16 = jnp.bfloat16
    xb = x[:, 0].astype(bf16)
    pairs = jnp.stack([xb[:, 0:2], xb[:, 2:4], xb[:, 4:6]], axis=0).reshape(NS, 2, L)
    xpad = jnp.pad(pairs, ((0, 0), (0, 0), (0, PH * M_IN - L)))
    x120 = xpad.reshape(NS, 2, M_IN, PH).transpose(0, 1, 3, 2).reshape(NS, 2 * PH, M_IN)

    w1 = jnp.concatenate([w1q[0], w1q[1], w1q[2]], axis=1).astype(bf16)
    wc2b, wf0b, wf1b = wc2.astype(bf16), wf0.astype(bf16), wf1.astype(bf16)
    s0b, s1b = s0.astype(bf16), s1.astype(bf16)
    nc = wout.shape[1]

    z, x44s = pl.pallas_call(
        _ae_kernel,
        out_shape=(jax.ShapeDtypeStruct((NS, 32, W4), bf16),
                   jax.ShapeDtypeStruct((NS, 32, 27), bf16)),
        grid=(NS // G,),
        in_specs=[
            pl.BlockSpec((G, 2 * PH, M_IN), lambda g: (g, 0, 0)),
            pl.BlockSpec((U1 * 16, 3 * 2 * PH), lambda g: (0, 0)),
            pl.BlockSpec((U1 * 16, 1), lambda g: (0, 0)),
            pl.BlockSpec((64, 64), lambda g: (0, 0)),
            pl.BlockSpec((64, 1), lambda g: (0, 0)),
            pl.BlockSpec((32, 32), lambda g: (0, 0)),
            pl.BlockSpec((32, 32), lambda g: (0, 0)),
            pl.BlockSpec((32, 1), lambda g: (0, 0)),
            pl.BlockSpec((W4, 27), lambda g: (0, 0)),
            pl.BlockSpec((W4, 27), lambda g: (0, 0)),
        ],
        out_specs=(pl.BlockSpec((G, 32, W4), lambda g: (g, 0, 0)),
                   pl.BlockSpec((G, 32, 27), lambda g: (g, 0, 0))),
        compiler_params=pltpu.CompilerParams(dimension_semantics=("parallel",)),
    )(x120, w1, b1s, wc2b, b2s, wf0b, wf1b, bf, s0b, s1b)

    z_flat = z.reshape(NS, 32 * W4)
    x44_flat = x44s.reshape(3, B, 32, 27).transpose(1, 2, 0, 3).reshape(B, 32 * W4)

    out, x6 = pl.pallas_call(
        _head_kernel,
        out_shape=(jax.ShapeDtypeStruct((B, nc), jnp.float32),
                   jax.ShapeDtypeStruct((B, 128), jnp.float32)),
        grid=(1,),
        in_specs=[
            pl.BlockSpec((NS, 32 * W4), lambda i: (0, 0)),
            pl.BlockSpec((B, 32 * W4), lambda i: (0, 0)),
            pl.BlockSpec((32 * W4, 32), lambda i: (0, 0)),
            pl.BlockSpec((1, 32), lambda i: (0, 0)),
            pl.BlockSpec((32 * W4, 32), lambda i: (0, 0)),
            pl.BlockSpec((1, 32), lambda i: (0, 0)),
            pl.BlockSpec((128, nc), lambda i: (0, 0)),
            pl.BlockSpec((1, nc), lambda i: (0, 0)),
        ],
        out_specs=(pl.BlockSpec((B, nc), lambda i: (0, 0)),
                   pl.BlockSpec((B, 128), lambda i: (0, 0))),
    )(z_flat, x44_flat, wcon.astype(bf16), bcon, wfl.astype(bf16), bfl,
      wout.astype(bf16), bout)
    return out, x6


# speedup vs baseline: 1.5881x; 1.2436x over previous
import jax
import jax.numpy as jnp
from jax.experimental import pallas as pl
from jax.experimental.pallas import tpu as pltpu

L_IN = 10178
PH = 120
M_PAD = 91
U1 = 24
W4 = 81
G = 4
_POOL_W = ((1, 88), (2, 87), (4, 86), (8, 85), (16, 84), (32, 82))


def _ae_kernel(x_ref, w1_ref, b1_ref, wc2_ref, b2_ref,
               wf0_ref, wf1_ref, bf_ref, s0_ref, s1_ref,
               z_ref, x44_ref):
    for i in range(G):
        xr = x_ref[i]
        x0, x1 = xr[0], xr[1]
        xs = jnp.concatenate([x0[0:89], x0[1:90], x0[2:91],
                              x1[0:89], x1[1:90], x1[2:91]], axis=1)
        m = jnp.dot(xs, w1_ref[...], preferred_element_type=jnp.float32) + b1_ref[...]

        for s, w in _POOL_W:
            a0, r = s // U1, s % U1
            sh = jnp.concatenate([m[a0:a0 + w, 16 * r:U1 * 16],
                                  m[a0 + 1:a0 + 1 + w, 0:16 * r]], axis=1)
            m = jnp.maximum(m[0:w, :], sh)
        p1 = jnp.concatenate([m[:, 0:16], m[:, 48:64],
                              m[:, 192:208], m[:, 240:256]], axis=1)
        p1 = jnp.maximum(p1, 0.0).astype(jnp.bfloat16)

        c2 = jnp.dot(p1, wc2_ref[...], preferred_element_type=jnp.float32) + b2_ref[...]
        z = jnp.maximum(jnp.maximum(c2[0:W4, 0:32], c2[0:W4, 32:64]),
                        c2[1:W4 + 1, 0:32])
        z = jnp.maximum(z, 0.0)
        zb = z.astype(jnp.bfloat16)
        z_ref[i] = zb

        t0 = jnp.dot(s0_ref[...], zb, preferred_element_type=jnp.float32)
        t1 = jnp.dot(s1_ref[...], zb, preferred_element_type=jnp.float32)
        slab = (jnp.dot(t0.astype(jnp.bfloat16), wf0_ref[...],
                        preferred_element_type=jnp.float32)
                + jnp.dot(t1.astype(jnp.bfloat16), wf1_ref[...],
                          preferred_element_type=jnp.float32)
                + bf_ref[...])
        x44_ref[i] = jnp.maximum(slab, 0.0).astype(jnp.bfloat16)


def _head_kernel(zf_ref, x44f_ref, wcon_ref, bcon_ref, wfl_ref, bfl_ref,
                 wout_ref, bout_ref, out_ref, x6_ref):
    B = x44f_ref.shape[0]
    x5 = jnp.dot(zf_ref[...], wcon_ref[...],
                 preferred_element_type=jnp.float32) + bcon_ref[...]
    x5 = jnp.maximum(x5, 0.0)
    x54 = jnp.dot(x44f_ref[...], wfl_ref[...],
                  preferred_element_type=jnp.float32) + bfl_ref[...]
    x54 = jnp.maximum(x54, 0.0)
    x6 = jnp.concatenate([x5[0:B], x5[B:2 * B], x5[2 * B:3 * B], x54], axis=1)
    x6_ref[...] = x6
    out_ref[...] = jnp.dot(x6.astype(jnp.bfloat16), wout_ref[...],
                           preferred_element_type=jnp.float32) + bout_ref[...]


def kernel(x, w1q, b1s, wc2, b2s, wf0, wf1, bf, s0, s1,
           wcon, bcon, wfl, bfl, wout, bout):
    B, C, S, L = x.shape
    NS = 3 * B
    BG = B // G
    bf16 = jnp.bfloat16

    xb = jnp.pad(x[:, 0].astype(bf16), ((0, 0), (0, 0), (0, M_PAD * PH - L)))
    xm = xb.reshape(B, S, M_PAD, PH)

    w1 = jnp.concatenate(
        [w1q[q][:, c * PH:(c + 1) * PH] for c in range(2) for q in range(3)],
        axis=1).T.astype(bf16)
    b1t = b1s.reshape(1, U1 * 16)
    wc2t, b2t = wc2.T.astype(bf16), b2s.reshape(1, 64)
    wf0t, wf1t = wf0.T.astype(bf16), wf1.T.astype(bf16)
    s0t, s1t = s0.T.astype(bf16), s1.T.astype(bf16)
    bft = bf.reshape(1, 32)
    nc = wout.shape[1]

    zt, x44t = pl.pallas_call(
        _ae_kernel,
        out_shape=(jax.ShapeDtypeStruct((NS, W4, 32), bf16),
                   jax.ShapeDtypeStruct((NS, 27, 32), bf16)),
        grid=(NS // G,),
        in_specs=[
            pl.BlockSpec((G, 2, M_PAD, PH), lambda g, BG=BG: (g % BG, g // BG, 0, 0)),
            pl.BlockSpec((3 * 2 * PH, U1 * 16), lambda g: (0, 0)),
            pl.BlockSpec((1, U1 * 16), lambda g: (0, 0)),
            pl.BlockSpec((64, 64), lambda g: (0, 0)),
            pl.BlockSpec((1, 64), lambda g: (0, 0)),
            pl.BlockSpec((32, 32), lambda g: (0, 0)),
            pl.BlockSpec((32, 32), lambda g: (0, 0)),
            pl.BlockSpec((1, 32), lambda g: (0, 0)),
            pl.BlockSpec((27, W4), lambda g: (0, 0)),
            pl.BlockSpec((27, W4), lambda g: (0, 0)),
        ],
        out_specs=(pl.BlockSpec((G, W4, 32), lambda g: (g, 0, 0)),
                   pl.BlockSpec((G, 27, 32), lambda g: (g, 0, 0))),
        compiler_params=pltpu.CompilerParams(dimension_semantics=("parallel",)),
    )(xm, w1, b1t, wc2t, b2t, wf0t, wf1t, bft, s0t, s1t)

    z_flat = zt.reshape(NS, W4 * 32)
    x44_flat = x44t.reshape(3, B, 27 * 32).transpose(1, 0, 2).reshape(B, W4 * 32)
    wcon_p = wcon.reshape(32, W4, 32).transpose(1, 0, 2).reshape(W4 * 32, 32)
    wfl_p = wfl.reshape(32, 3, 27, 32).transpose(1, 2, 0, 3).reshape(W4 * 32, 32)

    out, x6 = pl.pallas_call(
        _head_kernel,
        out_shape=(jax.ShapeDtypeStruct((B, nc), jnp.float32),
                   jax.ShapeDtypeStruct((B, 128), jnp.float32)),
        grid=(1,),
        in_specs=[
            pl.BlockSpec((NS, W4 * 32), lambda i: (0, 0)),
            pl.BlockSpec((B, W4 * 32), lambda i: (0, 0)),
            pl.BlockSpec((W4 * 32, 32), lambda i: (0, 0)),
            pl.BlockSpec((1, 32), lambda i: (0, 0)),
            pl.BlockSpec((W4 * 32, 32), lambda i: (0, 0)),
            pl.BlockSpec((1, 32), lambda i: (0, 0)),
            pl.BlockSpec((128, nc), lambda i: (0, 0)),
            pl.BlockSpec((1, nc), lambda i: (0, 0)),
        ],
        out_specs=(pl.BlockSpec((B, nc), lambda i: (0, 0)),
                   pl.BlockSpec((B, 128), lambda i: (0, 0))),
    )(z_flat, x44_flat, wcon_p.astype(bf16), bcon, wfl_p.astype(bf16), bfl,
      wout.astype(bf16), bout)
    return out, x6
